# Initial kernel scaffold; baseline (speedup 1.0000x reference)
import jax
import jax.numpy as jnp
from jax.experimental import pallas as pl


def kernel(x, prompt_add, prompt_mul, W_ka, b_ka, W_va, b_va, W_km, b_km, W_vm, b_vm):
    raise NotImplementedError("write your pallas kernel here")


# speedup vs baseline: 18.7449x; 18.7449x over previous
import functools
import math

import jax
import jax.numpy as jnp
from jax.experimental import pallas as pl

_NEG = -3.0e38


def _block_kernel(x_ref, qat_ref, qmt_ref,
                  wka_ref, bka_ref, wva_ref, bva_ref,
                  wkm_ref, bkm_ref, wvm_ref, bvm_ref,
                  out_ref, *, bb, n, d, p, topk):
    xb = x_ref[...]

    def branch(x2, qt_ref, wk_ref, bk_ref, wv_ref, bv_ref):
        k = jax.lax.dot(x2, wk_ref[...]) + bk_ref[...]
        v = jax.lax.dot(x2, wv_ref[...]) + bv_ref[...]
        att = (jax.lax.dot(k, qt_ref[...]).reshape(bb, n, p)
               / math.sqrt(d))
        m0 = jnp.max(att, axis=1, keepdims=True)
        cur, thresh = att, m0
        for _ in range(topk - 1):
            cur = jnp.where(cur >= thresh, _NEG, cur)
            thresh = jnp.max(cur, axis=1, keepdims=True)
        e = jnp.where(att >= thresh, jnp.exp(att - m0), 0.0)
        w = e / jnp.sum(e, axis=1, keepdims=True)
        return jax.lax.dot_general(
            w, v.reshape(bb, n, d), (((1,), (1,)), ((0,), (0,))),
            preferred_element_type=jnp.float32)

    x2 = xb.reshape(bb * n, d)
    out_ref[:, :p, :] = branch(x2, qat_ref, wka_ref, bka_ref,
                               wva_ref, bva_ref)

    xl = jnp.log(jnp.maximum(xb, 0.0) + 1e-5).reshape(bb * n, d)
    out_mul_log = branch(xl, qmt_ref, wkm_ref, bkm_ref, wvm_ref, bvm_ref)
    out_ref[:, p:, :] = jnp.exp(jnp.minimum(out_mul_log, 10.0))


def kernel(x, prompt_add, prompt_mul, W_ka, b_ka, W_va, b_va,
           W_km, b_km, W_vm, b_vm):
    b, n, d = x.shape
    p = prompt_add.shape[1]
    topk = 8
    bb = 64
    while b % bb:
        bb //= 2

    qat = prompt_add.reshape(p, d).T
    qmt = prompt_mul.reshape(p, d).T

    full = lambda shape: pl.BlockSpec(shape, lambda i: (0,) * len(shape))
    out = pl.pallas_call(
        functools.partial(_block_kernel, bb=bb, n=n, d=d, p=p, topk=topk),
        grid=(b // bb,),
        in_specs=[
            pl.BlockSpec((bb, n, d), lambda i: (i, 0, 0)),
            full((d, p)), full((d, p)),
            full((d, d)), full((1, d)), full((d, d)), full((1, d)),
            full((d, d)), full((1, d)), full((d, d)), full((1, d)),
        ],
        out_specs=pl.BlockSpec((bb, 2 * p, d), lambda i: (i, 0, 0)),
        out_shape=jax.ShapeDtypeStruct((b, 2 * p, d), jnp.float32),
    )(x, qat, qmt,
      W_ka, b_ka.reshape(1, d), W_va, b_va.reshape(1, d),
      W_km, b_km.reshape(1, d), W_vm, b_vm.reshape(1, d))
    return out


# speedup vs baseline: 26.1863x; 1.3970x over previous
import functools
import math

import jax
import jax.numpy as jnp
from jax.experimental import pallas as pl

_NEG = -3.0e38


def _block_kernel(x_ref, qa_ref, qm_ref,
                  wka_ref, bka_ref, wva_ref, bva_ref,
                  wkm_ref, bkm_ref, wvm_ref, bvm_ref,
                  out_ref, *, bb, n, d, p, topk):
    xb = x_ref[...]

    def branch(x2, q_ref, wk_ref, bk_ref, wv_ref, bv_ref):
        k = (jax.lax.dot(x2, wk_ref[...]) + bk_ref[...]).reshape(bb, n, d)
        v = (jax.lax.dot(x2, wv_ref[...]) + bv_ref[...]).reshape(bb, n, d)
        qb = jnp.broadcast_to(q_ref[...].reshape(1, p, d), (bb, p, d))
        att = jax.lax.dot_general(
            qb, k, (((2,), (2,)), ((0,), (0,))),
            preferred_element_type=jnp.float32) / math.sqrt(d)
        m0 = jnp.max(att, axis=2, keepdims=True)
        cur, thresh = att, m0
        for _ in range(topk - 1):
            cur = jnp.where(cur >= thresh, _NEG, cur)
            thresh = jnp.max(cur, axis=2, keepdims=True)
        e = jnp.where(att >= thresh, jnp.exp(att - m0), 0.0)
        w = e / jnp.sum(e, axis=2, keepdims=True)
        return jax.lax.dot_general(
            w, v, (((2,), (1,)), ((0,), (0,))),
            preferred_element_type=jnp.float32)

    x2 = xb.reshape(bb * n, d)
    out_ref[:, :p, :] = branch(x2, qa_ref, wka_ref, bka_ref,
                               wva_ref, bva_ref)

    xl = jnp.log(jnp.maximum(xb, 0.0) + 1e-5).reshape(bb * n, d)
    out_mul_log = branch(xl, qm_ref, wkm_ref, bkm_ref, wvm_ref, bvm_ref)
    out_ref[:, p:, :] = jnp.exp(jnp.minimum(out_mul_log, 10.0))


def kernel(x, prompt_add, prompt_mul, W_ka, b_ka, W_va, b_va,
           W_km, b_km, W_vm, b_vm):
    b, n, d = x.shape
    p = prompt_add.shape[1]
    topk = 8
    bb = 64
    while b % bb:
        bb //= 2

    full = lambda shape: pl.BlockSpec(shape, lambda i: (0,) * len(shape))
    out = pl.pallas_call(
        functools.partial(_block_kernel, bb=bb, n=n, d=d, p=p, topk=topk),
        grid=(b // bb,),
        in_specs=[
            pl.BlockSpec((bb, n, d), lambda i: (i, 0, 0)),
            full((p, d)), full((p, d)),
            full((d, d)), full((1, d)), full((d, d)), full((1, d)),
            full((d, d)), full((1, d)), full((d, d)), full((1, d)),
        ],
        out_specs=pl.BlockSpec((bb, 2 * p, d), lambda i: (i, 0, 0)),
        out_shape=jax.ShapeDtypeStruct((b, 2 * p, d), jnp.float32),
    )(x, prompt_add.reshape(p, d), prompt_mul.reshape(p, d),
      W_ka, b_ka.reshape(1, d), W_va, b_va.reshape(1, d),
      W_km, b_km.reshape(1, d), W_vm, b_vm.reshape(1, d))
    return out


# speedup vs baseline: 26.2037x; 1.0007x over previous
import functools
import math

import jax
import jax.numpy as jnp
from jax.experimental import pallas as pl
from jax.experimental.pallas import tpu as pltpu

_NEG = -3.0e38


def _block_kernel(x_ref, qa_ref, qm_ref,
                  wka_ref, bka_ref, wva_ref, bva_ref,
                  wkm_ref, bkm_ref, wvm_ref, bvm_ref,
                  out_ref, *, bb, n, d, p, topk):
    xb = x_ref[...]

    def branch(x2, q_ref, wk_ref, bk_ref, wv_ref, bv_ref):
        k = (jax.lax.dot(x2, wk_ref[...]) + bk_ref[...]).reshape(bb, n, d)
        v = (jax.lax.dot(x2, wv_ref[...]) + bv_ref[...]).reshape(bb, n, d)
        qb = jnp.broadcast_to(q_ref[...].reshape(1, p, d), (bb, p, d))
        att = jax.lax.dot_general(
            qb, k, (((2,), (2,)), ((0,), (0,))),
            preferred_element_type=jnp.float32) / math.sqrt(d)
        m0 = jnp.max(att, axis=2, keepdims=True)
        thresh = m0
        for _ in range(topk - 1):
            thresh = jnp.max(jnp.where(att < thresh, att, _NEG),
                             axis=2, keepdims=True)
        e = jnp.where(att >= thresh, jnp.exp(att - m0), 0.0)
        w = e / jnp.sum(e, axis=2, keepdims=True)
        return jax.lax.dot_general(
            w, v, (((2,), (1,)), ((0,), (0,))),
            preferred_element_type=jnp.float32)

    x2 = xb.reshape(bb * n, d)
    out_ref[:, :p, :] = branch(x2, qa_ref, wka_ref, bka_ref,
                               wva_ref, bva_ref)

    xl = jnp.log(jnp.maximum(xb, 0.0) + 1e-5).reshape(bb * n, d)
    out_mul_log = branch(xl, qm_ref, wkm_ref, bkm_ref, wvm_ref, bvm_ref)
    out_ref[:, p:, :] = jnp.exp(jnp.minimum(out_mul_log, 10.0))


def kernel(x, prompt_add, prompt_mul, W_ka, b_ka, W_va, b_va,
           W_km, b_km, W_vm, b_vm):
    b, n, d = x.shape
    p = prompt_add.shape[1]
    topk = 8
    bb = 64
    while b % bb:
        bb //= 2

    full = lambda shape: pl.BlockSpec(shape, lambda i: (0,) * len(shape))
    out = pl.pallas_call(
        functools.partial(_block_kernel, bb=bb, n=n, d=d, p=p, topk=topk),
        grid=(b // bb,),
        in_specs=[
            pl.BlockSpec((bb, n, d), lambda i: (i, 0, 0)),
            full((p, d)), full((p, d)),
            full((d, d)), full((1, d)), full((d, d)), full((1, d)),
            full((d, d)), full((1, d)), full((d, d)), full((1, d)),
        ],
        out_specs=pl.BlockSpec((bb, 2 * p, d), lambda i: (i, 0, 0)),
        out_shape=jax.ShapeDtypeStruct((b, 2 * p, d), jnp.float32),
        compiler_params=pltpu.CompilerParams(
            dimension_semantics=("parallel",)),
    )(x, prompt_add.reshape(p, d), prompt_mul.reshape(p, d),
      W_ka, b_ka.reshape(1, d), W_va, b_va.reshape(1, d),
      W_km, b_km.reshape(1, d), W_vm, b_vm.reshape(1, d))
    return out


# speedup vs baseline: 26.5905x; 1.0148x over previous
import functools
import math

import jax
import jax.numpy as jnp
from jax.experimental import pallas as pl
from jax.experimental.pallas import tpu as pltpu

_NEG = -3.0e38


def _block_kernel(x_ref, qa_ref, qm_ref,
                  wka_ref, bka_ref, wva_ref, bva_ref,
                  wkm_ref, bkm_ref, wvm_ref, bvm_ref,
                  out_ref, *, bb, n, d, p, topk):
    xb = x_ref[...]

    def branch(x2, q_ref, wk_ref, bk_ref, wv_ref, bv_ref):
        k = (jax.lax.dot(x2, wk_ref[...]) + bk_ref[...]).reshape(bb, n, d)
        v = (jax.lax.dot(x2, wv_ref[...]) + bv_ref[...]).reshape(bb, n, d)
        qb = jnp.broadcast_to(q_ref[...].reshape(1, p, d), (bb, p, d))
        att = jax.lax.dot_general(
            qb, k, (((2,), (2,)), ((0,), (0,))),
            preferred_element_type=jnp.float32) / math.sqrt(d)
        m0 = jnp.max(att, axis=2, keepdims=True)
        thresh = m0
        for _ in range(topk - 1):
            thresh = jnp.max(jnp.where(att < thresh, att, _NEG),
                             axis=2, keepdims=True)
        e = jnp.where(att >= thresh, jnp.exp(att - m0), 0.0)
        w = e / jnp.sum(e, axis=2, keepdims=True)
        return jax.lax.dot_general(
            w, v, (((2,), (1,)), ((0,), (0,))),
            preferred_element_type=jnp.float32)

    x2 = xb.reshape(bb * n, d)
    out_ref[:, :p, :] = branch(x2, qa_ref, wka_ref, bka_ref,
                               wva_ref, bva_ref)

    xl = jnp.log(jnp.maximum(xb, 0.0) + 1e-5).reshape(bb * n, d)
    out_mul_log = branch(xl, qm_ref, wkm_ref, bkm_ref, wvm_ref, bvm_ref)
    out_ref[:, p:, :] = jnp.exp(jnp.minimum(out_mul_log, 10.0))


def kernel(x, prompt_add, prompt_mul, W_ka, b_ka, W_va, b_va,
           W_km, b_km, W_vm, b_vm):
    b, n, d = x.shape
    p = prompt_add.shape[1]
    topk = 8
    bb = 128
    while b % bb:
        bb //= 2

    full = lambda shape: pl.BlockSpec(shape, lambda i: (0,) * len(shape))
    out = pl.pallas_call(
        functools.partial(_block_kernel, bb=bb, n=n, d=d, p=p, topk=topk),
        grid=(b // bb,),
        in_specs=[
            pl.BlockSpec((bb, n, d), lambda i: (i, 0, 0)),
            full((p, d)), full((p, d)),
            full((d, d)), full((1, d)), full((d, d)), full((1, d)),
            full((d, d)), full((1, d)), full((d, d)), full((1, d)),
        ],
        out_specs=pl.BlockSpec((bb, 2 * p, d), lambda i: (i, 0, 0)),
        out_shape=jax.ShapeDtypeStruct((b, 2 * p, d), jnp.float32),
        compiler_params=pltpu.CompilerParams(
            dimension_semantics=("parallel",)),
    )(x, prompt_add.reshape(p, d), prompt_mul.reshape(p, d),
      W_ka, b_ka.reshape(1, d), W_va, b_va.reshape(1, d),
      W_km, b_km.reshape(1, d), W_vm, b_vm.reshape(1, d))
    return out
